# Initial kernel scaffold; baseline (speedup 1.0000x reference)
import jax
import jax.numpy as jnp
from jax.experimental import pallas as pl


def kernel(y_hs_true, y_hs_pred, y_hs_batch, y_pks_true, y_pks_pred, y_pks_batch, y_pI_true, y_pI_pred):
    raise NotImplementedError("write your pallas kernel here")


# speedup vs baseline: 1.6191x; 1.6191x over previous
import functools

import jax
import jax.numpy as jnp
from jax import lax
from jax.experimental import pallas as pl
from jax.experimental.pallas import tpu as pltpu
from jax.experimental.pallas import tpu_sc as plsc

EPS = 1e-06
N = 262144
B = 1024
R = 1024
G = N // R
NW = 32
CH = N // NW
HALF_LN_2PI = 0.9189385332046727


def _lgamma_pos(x):
    small = x < 5.0
    xs = jnp.where(small, x, 1.0)
    p = xs * (xs + 1.0) * (xs + 2.0) * (xs + 3.0) * (xs + 4.0)
    adj = jnp.where(small, jnp.log(p), 0.0)
    z = jnp.where(small, x + 5.0, x)
    r = 1.0 / z
    w = r * r
    return (z - 0.5) * jnp.log(z) - z + HALF_LN_2PI + r * (1.0 / 12.0 - w * (1.0 / 360.0)) - adj


def _k1_body(hs_true_ref, hs_pred_ref, pks_true_ref, pks_pred_ref,
             lhs_ref, lpks_ref):
    yp = hs_pred_ref[...]
    mu = 1.0 / (1.0 + jnp.exp(-yp[:, :51]))
    ph_raw = yp[:, 51:]
    phi = jnp.maximum(ph_raw, 0.0) + jnp.log(1.0 + jnp.exp(-jnp.abs(ph_raw)))
    y = jnp.clip(hs_true_ref[...], EPS, 1.0 - EPS)
    a = phi * mu
    b = (1.0 - mu) * phi
    loglik = (_lgamma_pos(phi + EPS)
              - _lgamma_pos(a + EPS)
              - _lgamma_pos(b + EPS)
              + a * jnp.log(y)
              + (b - 1.0) * jnp.log(1.0 - y))
    wneg = jnp.full((1, 51), -1.0 / 51.0, jnp.float32)
    lhs = lax.dot_general(wneg, loglik, (((1,), (1,)), ((), ())),
                          preferred_element_type=jnp.float32)
    lhs_ref[...] = lhs.reshape(1, 1, R)

    pp = pks_pred_ref[...]
    pmu = pp[:, :8]
    plv = pp[:, 8:] + EPS
    pt = pks_true_ref[...]
    lp = 0.5 * (plv + (pt - pmu) * (pt - pmu) * jnp.exp(-plv))
    ones = jnp.full((1, 8), 1.0, jnp.float32)
    lpk = lax.dot_general(ones, lp, (((1,), (1,)), ((), ())),
                          preferred_element_type=jnp.float32)
    lpks_ref[...] = lpk.reshape(1, 1, R)


def _row_losses(y_hs_true, y_hs_pred, y_pks_true, y_pks_pred):
    out = pl.pallas_call(
        _k1_body,
        grid=(G,),
        in_specs=[
            pl.BlockSpec((R, 51), lambda i: (i, 0)),
            pl.BlockSpec((R, 102), lambda i: (i, 0)),
            pl.BlockSpec((R, 8), lambda i: (i, 0)),
            pl.BlockSpec((R, 16), lambda i: (i, 0)),
        ],
        out_specs=[
            pl.BlockSpec((1, 1, R), lambda i: (i, 0, 0)),
            pl.BlockSpec((1, 1, R), lambda i: (i, 0, 0)),
        ],
        out_shape=[
            jax.ShapeDtypeStruct((G, 1, R), jnp.float32),
            jax.ShapeDtypeStruct((G, 1, R), jnp.float32),
        ],
    )(y_hs_true, y_hs_pred, y_pks_true, y_pks_pred)
    return out[0].reshape(N), out[1].reshape(N)


def _seg_accumulate(vals_ref, ids_ref, acc_s_ref, acc_c_ref):
    lane_base = lax.iota(jnp.int32, 16) * B
    vones = jnp.full((16,), 1.0, jnp.float32)

    def body(i, carry):
        d = ids_ref[pl.ds(i * 16, 16)]
        v = vals_ref[pl.ds(i * 16, 16)]
        idx = lane_base + d
        plsc.addupdate_scatter(acc_s_ref, [idx], v)
        plsc.addupdate_scatter(acc_c_ref, [idx], vones)
        return carry

    lax.fori_loop(0, CH // 16, body, 0)


def _k2_body(lhs_hbm, hs_ids_hbm, lpks_hbm, pks_ids_hbm, parts_hbm,
             vals_v, ids_v, acc_hs_s, acc_hs_c, acc_pks_s, acc_pks_c):
    wid = lax.axis_index("s") * 2 + lax.axis_index("c")
    base = wid * CH
    z16 = jnp.zeros((16,), jnp.float32)

    def zbody(j, c):
        acc_hs_s[pl.ds(j * 16, 16)] = z16
        acc_hs_c[pl.ds(j * 16, 16)] = z16
        acc_pks_s[pl.ds(j * 16, 16)] = z16
        acc_pks_c[pl.ds(j * 16, 16)] = z16
        return c

    lax.fori_loop(0, 16 * B // 16, zbody, 0)

    pltpu.sync_copy(lhs_hbm.at[pl.ds(base, CH)], vals_v)
    pltpu.sync_copy(hs_ids_hbm.at[pl.ds(base, CH)], ids_v)
    _seg_accumulate(vals_v, ids_v, acc_hs_s, acc_hs_c)

    pltpu.sync_copy(lpks_hbm.at[pl.ds(base, CH)], vals_v)
    pltpu.sync_copy(pks_ids_hbm.at[pl.ds(base, CH)], ids_v)
    _seg_accumulate(vals_v, ids_v, acc_pks_s, acc_pks_c)

    pltpu.sync_copy(acc_hs_s, parts_hbm.at[wid])
    pltpu.sync_copy(acc_hs_c, parts_hbm.at[NW + wid])
    pltpu.sync_copy(acc_pks_s, parts_hbm.at[2 * NW + wid])
    pltpu.sync_copy(acc_pks_c, parts_hbm.at[3 * NW + wid])


def _segment_partials(l_hs, hs_ids, l_pks, pks_ids):
    mesh = plsc.VectorSubcoreMesh(core_axis_name="c", subcore_axis_name="s")
    k2 = functools.partial(
        pl.kernel,
        mesh=mesh,
        compiler_params=pltpu.CompilerParams(needs_layout_passes=False),
        out_type=jax.ShapeDtypeStruct((4 * NW, 16 * B), jnp.float32),
        scratch_types=[
            pltpu.VMEM((CH,), jnp.float32),
            pltpu.VMEM((CH,), jnp.int32),
            pltpu.VMEM((16 * B,), jnp.float32),
            pltpu.VMEM((16 * B,), jnp.float32),
            pltpu.VMEM((16 * B,), jnp.float32),
            pltpu.VMEM((16 * B,), jnp.float32),
        ],
    )(_k2_body)
    return k2(l_hs, hs_ids, l_pks, pks_ids)


def _k3_body(parts_ref, pI_true_ref, pI_pred_ref, out_ref):
    P = parts_ref[...]
    W = NW * 16
    hs_s = jnp.sum(P[0:W], axis=0, keepdims=True)
    hs_c = jnp.sum(P[W:2 * W], axis=0, keepdims=True)
    pks_s = jnp.sum(P[2 * W:3 * W], axis=0, keepdims=True)
    pks_c = jnp.sum(P[3 * W:4 * W], axis=0, keepdims=True)
    hs_m = jnp.sum(hs_s / jnp.maximum(hs_c, 1.0)) / B
    pks_m = jnp.sum(pks_s / jnp.maximum(pks_c, 1.0)) / B

    pp = pI_pred_ref[...]
    pmu = pp[:, :8]
    plv = pp[:, 8:] + EPS
    pt = pI_true_ref[...]
    lp = 0.5 * (plv + (pt - pmu) * (pt - pmu) * jnp.exp(-plv))
    pI_m = jnp.sum(lp) / B

    sub = lax.broadcasted_iota(jnp.int32, (8, 128), 0)
    lane = lax.broadcasted_iota(jnp.int32, (8, 128), 1)
    out = jnp.where((sub == 0) & (lane == 0), hs_m, 0.0)
    out = out + jnp.where((sub == 0) & (lane == 1), pks_m, 0.0)
    out = out + jnp.where((sub == 0) & (lane == 2), pI_m, 0.0)
    out_ref[...] = out


def _combine(parts, y_pI_true, y_pI_pred):
    return pl.pallas_call(
        _k3_body,
        out_shape=jax.ShapeDtypeStruct((8, 128), jnp.float32),
    )(parts, y_pI_true, y_pI_pred)


def kernel(y_hs_true, y_hs_pred, y_hs_batch, y_pks_true, y_pks_pred,
           y_pks_batch, y_pI_true, y_pI_pred):
    l_hs, l_pks = _row_losses(y_hs_true, y_hs_pred, y_pks_true, y_pks_pred)
    parts = _segment_partials(l_hs, y_hs_batch.astype(jnp.int32),
                              l_pks, y_pks_batch.astype(jnp.int32))
    out = _combine(parts.reshape(4 * NW * 16, B), y_pI_true, y_pI_pred)
    return (out[0, :3], jnp.zeros(1), jnp.zeros(1))


# speedup vs baseline: 1.6326x; 1.0083x over previous
import functools

import jax
import jax.numpy as jnp
from jax import lax
from jax.experimental import pallas as pl
from jax.experimental.pallas import tpu as pltpu
from jax.experimental.pallas import tpu_sc as plsc

EPS = 1e-06
N = 262144
B = 1024
R = 1024
G = N // R
NW = 32
CH = N // NW
HALF_LN_2PI = 0.9189385332046727


def _lgamma_pos(x):
    small = x < 5.0
    xs = jnp.where(small, x, 1.0)
    p = xs * (xs + 1.0) * (xs + 2.0) * (xs + 3.0) * (xs + 4.0)
    adj = jnp.where(small, jnp.log(p), 0.0)
    z = jnp.where(small, x + 5.0, x)
    r = 1.0 / z
    w = r * r
    return (z - 0.5) * jnp.log(z) - z + HALF_LN_2PI + r * (1.0 / 12.0 - w * (1.0 / 360.0)) - adj


def _k1_body(hs_true_ref, hs_pred_ref, pks_true_ref, pks_pred_ref,
             lhs_ref, lpks_ref):
    yp = hs_pred_ref[...]
    mu = 1.0 / (1.0 + jnp.exp(-yp[:, :51]))
    ph_raw = yp[:, 51:]
    phi = jnp.maximum(ph_raw, 0.0) + jnp.log(1.0 + jnp.exp(-jnp.abs(ph_raw)))
    y = jnp.clip(hs_true_ref[...], EPS, 1.0 - EPS)
    a = phi * mu
    b = (1.0 - mu) * phi
    loglik = (_lgamma_pos(phi + EPS)
              - _lgamma_pos(a + EPS)
              - _lgamma_pos(b + EPS)
              + a * jnp.log(y)
              + (b - 1.0) * jnp.log(1.0 - y))
    wneg = jnp.full((1, 51), -1.0 / 51.0, jnp.float32)
    lhs = lax.dot_general(wneg, loglik, (((1,), (1,)), ((), ())),
                          preferred_element_type=jnp.float32)
    lhs_ref[...] = lhs.reshape(R)

    pp = pks_pred_ref[...]
    pmu = pp[:, :8]
    plv = pp[:, 8:] + EPS
    pt = pks_true_ref[...]
    lp = 0.5 * (plv + (pt - pmu) * (pt - pmu) * jnp.exp(-plv))
    ones = jnp.full((1, 8), 1.0, jnp.float32)
    lpk = lax.dot_general(ones, lp, (((1,), (1,)), ((), ())),
                          preferred_element_type=jnp.float32)
    lpks_ref[...] = lpk.reshape(R)


def _row_losses(y_hs_true, y_hs_pred, y_pks_true, y_pks_pred):
    out = pl.pallas_call(
        _k1_body,
        grid=(G,),
        in_specs=[
            pl.BlockSpec((R, 51), lambda i: (i, 0)),
            pl.BlockSpec((R, 102), lambda i: (i, 0)),
            pl.BlockSpec((R, 8), lambda i: (i, 0)),
            pl.BlockSpec((R, 16), lambda i: (i, 0)),
        ],
        out_specs=[
            pl.BlockSpec((R,), lambda i: (i,)),
            pl.BlockSpec((R,), lambda i: (i,)),
        ],
        out_shape=[
            jax.ShapeDtypeStruct((N,), jnp.float32),
            jax.ShapeDtypeStruct((N,), jnp.float32),
        ],
    )(y_hs_true, y_hs_pred, y_pks_true, y_pks_pred)
    return out[0], out[1]


def _seg_accumulate(vals_ref, ids_ref, acc_s_ref, acc_c_ref):
    lane = lax.iota(jnp.int32, 16)
    vones = jnp.full((16,), 1.0, jnp.float32)

    def body(i, carry):
        d = ids_ref[pl.ds(i * 16, 16)]
        v = vals_ref[pl.ds(i * 16, 16)]
        plsc.addupdate_scatter(acc_s_ref, [lane, d], v)
        plsc.addupdate_scatter(acc_c_ref, [lane, d], vones)
        return carry

    lax.fori_loop(0, CH // 16, body, 0)


def _k2_body(lhs_hbm, hs_ids_hbm, lpks_hbm, pks_ids_hbm, parts_hbm,
             vals_v, ids_v, acc_hs_s, acc_hs_c, acc_pks_s, acc_pks_c):
    wid = lax.axis_index("s") * 2 + lax.axis_index("c")
    base = wid * CH
    z16 = jnp.zeros((16,), jnp.float32)

    def zbody(j, c):
        r = j // (B // 16)
        col = (j % (B // 16)) * 16
        acc_hs_s[r, pl.ds(col, 16)] = z16
        acc_hs_c[r, pl.ds(col, 16)] = z16
        acc_pks_s[r, pl.ds(col, 16)] = z16
        acc_pks_c[r, pl.ds(col, 16)] = z16
        return c

    lax.fori_loop(0, 16 * (B // 16), zbody, 0)

    pltpu.sync_copy(lhs_hbm.at[pl.ds(base, CH)], vals_v)
    pltpu.sync_copy(hs_ids_hbm.at[pl.ds(base, CH)], ids_v)
    _seg_accumulate(vals_v, ids_v, acc_hs_s, acc_hs_c)

    pltpu.sync_copy(lpks_hbm.at[pl.ds(base, CH)], vals_v)
    pltpu.sync_copy(pks_ids_hbm.at[pl.ds(base, CH)], ids_v)
    _seg_accumulate(vals_v, ids_v, acc_pks_s, acc_pks_c)

    pltpu.sync_copy(acc_hs_s, parts_hbm.at[wid])
    pltpu.sync_copy(acc_hs_c, parts_hbm.at[NW + wid])
    pltpu.sync_copy(acc_pks_s, parts_hbm.at[2 * NW + wid])
    pltpu.sync_copy(acc_pks_c, parts_hbm.at[3 * NW + wid])


def _segment_partials(l_hs, hs_ids, l_pks, pks_ids):
    mesh = plsc.VectorSubcoreMesh(core_axis_name="c", subcore_axis_name="s")
    k2 = functools.partial(
        pl.kernel,
        mesh=mesh,
        compiler_params=pltpu.CompilerParams(needs_layout_passes=False),
        out_type=jax.ShapeDtypeStruct((4 * NW, 16, B), jnp.float32),
        scratch_types=[
            pltpu.VMEM((CH,), jnp.float32),
            pltpu.VMEM((CH,), jnp.int32),
            pltpu.VMEM((16, B), jnp.float32),
            pltpu.VMEM((16, B), jnp.float32),
            pltpu.VMEM((16, B), jnp.float32),
            pltpu.VMEM((16, B), jnp.float32),
        ],
    )(_k2_body)
    return k2(l_hs, hs_ids, l_pks, pks_ids)


def _k3_body(parts_ref, pI_true_ref, pI_pred_ref, out_ref):
    P = parts_ref[...]
    hs_s = jnp.sum(P[0:NW], axis=(0, 1), keepdims=False)[None, :]
    hs_c = jnp.sum(P[NW:2 * NW], axis=(0, 1), keepdims=False)[None, :]
    pks_s = jnp.sum(P[2 * NW:3 * NW], axis=(0, 1), keepdims=False)[None, :]
    pks_c = jnp.sum(P[3 * NW:4 * NW], axis=(0, 1), keepdims=False)[None, :]
    hs_m = jnp.sum(hs_s / jnp.maximum(hs_c, 1.0)) / B
    pks_m = jnp.sum(pks_s / jnp.maximum(pks_c, 1.0)) / B

    pp = pI_pred_ref[...]
    pmu = pp[:, :8]
    plv = pp[:, 8:] + EPS
    pt = pI_true_ref[...]
    lp = 0.5 * (plv + (pt - pmu) * (pt - pmu) * jnp.exp(-plv))
    pI_m = jnp.sum(lp) / B

    sub = lax.broadcasted_iota(jnp.int32, (8, 128), 0)
    lane = lax.broadcasted_iota(jnp.int32, (8, 128), 1)
    out = jnp.where((sub == 0) & (lane == 0), hs_m, 0.0)
    out = out + jnp.where((sub == 0) & (lane == 1), pks_m, 0.0)
    out = out + jnp.where((sub == 0) & (lane == 2), pI_m, 0.0)
    out_ref[...] = out


def _combine(parts, y_pI_true, y_pI_pred):
    return pl.pallas_call(
        _k3_body,
        out_shape=jax.ShapeDtypeStruct((8, 128), jnp.float32),
    )(parts, y_pI_true, y_pI_pred)


def kernel(y_hs_true, y_hs_pred, y_hs_batch, y_pks_true, y_pks_pred,
           y_pks_batch, y_pI_true, y_pI_pred):
    l_hs, l_pks = _row_losses(y_hs_true, y_hs_pred, y_pks_true, y_pks_pred)
    parts = _segment_partials(l_hs, y_hs_batch.astype(jnp.int32),
                              l_pks, y_pks_batch.astype(jnp.int32))
    out = _combine(parts, y_pI_true, y_pI_pred)
    return (out[0, :3], jnp.zeros(1), jnp.zeros(1))


# speedup vs baseline: 3.8210x; 2.3405x over previous
import functools

import jax
import jax.numpy as jnp
from jax import lax
from jax.experimental import pallas as pl
from jax.experimental.pallas import tpu as pltpu
from jax.experimental.pallas import tpu_sc as plsc

EPS = 1e-06
N = 262144
B = 1024
R = 1024
G = N // R
NW = 32
CH = N // NW
HALF_LN_2PI = 0.9189385332046727


def _lgamma_pos(x):
    small = x < 5.0
    xs = jnp.where(small, x, 1.0)
    p = xs * (xs + 1.0) * (xs + 2.0) * (xs + 3.0) * (xs + 4.0)
    adj = jnp.where(small, jnp.log(p), 0.0)
    z = jnp.where(small, x + 5.0, x)
    r = 1.0 / z
    w = r * r
    return (z - 0.5) * jnp.log(z) - z + HALF_LN_2PI + r * (1.0 / 12.0 - w * (1.0 / 360.0)) - adj


def _k1_body(hs_true_ref, hs_pred_ref, pks_true_ref, pks_pred_ref,
             lhs_ref, lpks_ref):
    yp = hs_pred_ref[...]
    mu = 1.0 / (1.0 + jnp.exp(-yp[:51]))
    ph_raw = yp[51:]
    phi = jnp.maximum(ph_raw, 0.0) + jnp.log(1.0 + jnp.exp(-jnp.abs(ph_raw)))
    y = jnp.clip(hs_true_ref[...], EPS, 1.0 - EPS)
    a = phi * mu
    b = (1.0 - mu) * phi
    loglik = (_lgamma_pos(phi + EPS)
              - _lgamma_pos(a + EPS)
              - _lgamma_pos(b + EPS)
              + a * jnp.log(y)
              + (b - 1.0) * jnp.log(1.0 - y))
    wneg = jnp.full((1, 51), -1.0 / 51.0, jnp.float32)
    lhs = lax.dot_general(wneg, loglik, (((1,), (0,)), ((), ())),
                          preferred_element_type=jnp.float32)
    lhs_ref[...] = lhs.reshape(R)

    pp = pks_pred_ref[...]
    pmu = pp[:8]
    plv = pp[8:] + EPS
    pt = pks_true_ref[...]
    lp = 0.5 * (plv + (pt - pmu) * (pt - pmu) * jnp.exp(-plv))
    ones = jnp.full((1, 8), 1.0, jnp.float32)
    lpk = lax.dot_general(ones, lp, (((1,), (0,)), ((), ())),
                          preferred_element_type=jnp.float32)
    lpks_ref[...] = lpk.reshape(R)


def _row_losses(y_hs_true, y_hs_pred, y_pks_true, y_pks_pred):
    out = pl.pallas_call(
        _k1_body,
        grid=(G,),
        in_specs=[
            pl.BlockSpec((51, R), lambda i: (0, i)),
            pl.BlockSpec((102, R), lambda i: (0, i)),
            pl.BlockSpec((8, R), lambda i: (0, i)),
            pl.BlockSpec((16, R), lambda i: (0, i)),
        ],
        out_specs=[
            pl.BlockSpec((R,), lambda i: (i,)),
            pl.BlockSpec((R,), lambda i: (i,)),
        ],
        out_shape=[
            jax.ShapeDtypeStruct((N,), jnp.float32),
            jax.ShapeDtypeStruct((N,), jnp.float32),
        ],
    )(y_hs_true.T, y_hs_pred.T, y_pks_true.T, y_pks_pred.T)
    return out[0], out[1]


def _seg_accumulate(vals_ref, ids_ref, acc_s_ref, acc_c_ref):
    lane = lax.iota(jnp.int32, 16)
    vones = jnp.full((16,), 1.0, jnp.float32)

    def body(i, carry):
        d = ids_ref[pl.ds(i * 16, 16)]
        v = vals_ref[pl.ds(i * 16, 16)]
        plsc.addupdate_scatter(acc_s_ref, [lane, d], v)
        plsc.addupdate_scatter(acc_c_ref, [lane, d], vones)
        return carry

    lax.fori_loop(0, CH // 16, body, 0)


def _k2_body(lhs_hbm, hs_ids_hbm, lpks_hbm, pks_ids_hbm, parts_hbm,
             vals_v, ids_v, acc_hs_s, acc_hs_c, acc_pks_s, acc_pks_c):
    wid = lax.axis_index("s") * 2 + lax.axis_index("c")
    base = wid * CH
    z16 = jnp.zeros((16,), jnp.float32)

    def zbody(j, c):
        r = j // (B // 16)
        col = (j % (B // 16)) * 16
        acc_hs_s[r, pl.ds(col, 16)] = z16
        acc_hs_c[r, pl.ds(col, 16)] = z16
        acc_pks_s[r, pl.ds(col, 16)] = z16
        acc_pks_c[r, pl.ds(col, 16)] = z16
        return c

    lax.fori_loop(0, 16 * (B // 16), zbody, 0)

    pltpu.sync_copy(lhs_hbm.at[pl.ds(base, CH)], vals_v)
    pltpu.sync_copy(hs_ids_hbm.at[pl.ds(base, CH)], ids_v)
    _seg_accumulate(vals_v, ids_v, acc_hs_s, acc_hs_c)

    pltpu.sync_copy(lpks_hbm.at[pl.ds(base, CH)], vals_v)
    pltpu.sync_copy(pks_ids_hbm.at[pl.ds(base, CH)], ids_v)
    _seg_accumulate(vals_v, ids_v, acc_pks_s, acc_pks_c)

    pltpu.sync_copy(acc_hs_s, parts_hbm.at[wid])
    pltpu.sync_copy(acc_hs_c, parts_hbm.at[NW + wid])
    pltpu.sync_copy(acc_pks_s, parts_hbm.at[2 * NW + wid])
    pltpu.sync_copy(acc_pks_c, parts_hbm.at[3 * NW + wid])


def _segment_partials(l_hs, hs_ids, l_pks, pks_ids):
    mesh = plsc.VectorSubcoreMesh(core_axis_name="c", subcore_axis_name="s")
    k2 = functools.partial(
        pl.kernel,
        mesh=mesh,
        compiler_params=pltpu.CompilerParams(needs_layout_passes=False),
        out_type=jax.ShapeDtypeStruct((4 * NW, 16, B), jnp.float32),
        scratch_types=[
            pltpu.VMEM((CH,), jnp.float32),
            pltpu.VMEM((CH,), jnp.int32),
            pltpu.VMEM((16, B), jnp.float32),
            pltpu.VMEM((16, B), jnp.float32),
            pltpu.VMEM((16, B), jnp.float32),
            pltpu.VMEM((16, B), jnp.float32),
        ],
    )(_k2_body)
    return k2(l_hs, hs_ids, l_pks, pks_ids)


def _k3_body(parts_ref, pI_true_ref, pI_pred_ref, out_ref):
    P = parts_ref[...]
    hs_s = jnp.sum(P[0:NW], axis=(0, 1), keepdims=False)[None, :]
    hs_c = jnp.sum(P[NW:2 * NW], axis=(0, 1), keepdims=False)[None, :]
    pks_s = jnp.sum(P[2 * NW:3 * NW], axis=(0, 1), keepdims=False)[None, :]
    pks_c = jnp.sum(P[3 * NW:4 * NW], axis=(0, 1), keepdims=False)[None, :]
    hs_m = jnp.sum(hs_s / jnp.maximum(hs_c, 1.0)) / B
    pks_m = jnp.sum(pks_s / jnp.maximum(pks_c, 1.0)) / B

    pp = pI_pred_ref[...]
    pmu = pp[:8]
    plv = pp[8:] + EPS
    pt = pI_true_ref[...]
    lp = 0.5 * (plv + (pt - pmu) * (pt - pmu) * jnp.exp(-plv))
    pI_m = jnp.sum(lp) / B

    sub = lax.broadcasted_iota(jnp.int32, (8, 128), 0)
    lane = lax.broadcasted_iota(jnp.int32, (8, 128), 1)
    out = jnp.where((sub == 0) & (lane == 0), hs_m, 0.0)
    out = out + jnp.where((sub == 0) & (lane == 1), pks_m, 0.0)
    out = out + jnp.where((sub == 0) & (lane == 2), pI_m, 0.0)
    out_ref[...] = out


def _combine(parts, y_pI_true, y_pI_pred):
    return pl.pallas_call(
        _k3_body,
        out_shape=jax.ShapeDtypeStruct((8, 128), jnp.float32),
    )(parts, y_pI_true.T, y_pI_pred.T)


def kernel(y_hs_true, y_hs_pred, y_hs_batch, y_pks_true, y_pks_pred,
           y_pks_batch, y_pI_true, y_pI_pred):
    l_hs, l_pks = _row_losses(y_hs_true, y_hs_pred, y_pks_true, y_pks_pred)
    parts = _segment_partials(l_hs, y_hs_batch.astype(jnp.int32),
                              l_pks, y_pks_batch.astype(jnp.int32))
    out = _combine(parts, y_pI_true, y_pI_pred)
    return (out[0, :3], jnp.zeros(1), jnp.zeros(1))
